# baseline (speedup 1.0000x reference)
import jax
import jax.numpy as jnp
from jax.experimental import pallas as pl
from jax.experimental.pallas import tpu as pltpu

_ROWS = 16384
_COLS = 4096
_BLOCK_ROWS = 256


def _body(x_ref, m_ref, sum_ref, cnt_ref):
    @pl.when(pl.program_id(0) == 0)
    def _init():
        sum_ref[...] = jnp.zeros((1, 1), jnp.float32)
        cnt_ref[...] = jnp.zeros((1, 1), jnp.float32)

    x = x_ref[...]
    m = m_ref[...]
    sum_ref[...] += jnp.sum(jnp.where(m, x, 0.0)).reshape(1, 1)
    cnt_ref[...] += jnp.sum(m.astype(jnp.float32)).reshape(1, 1)


def kernel(input, data_mask):
    grid = (_ROWS // _BLOCK_ROWS,)
    s, c = pl.pallas_call(
        _body,
        grid=grid,
        in_specs=[
            pl.BlockSpec((_BLOCK_ROWS, _COLS), lambda i: (i, 0)),
            pl.BlockSpec((_BLOCK_ROWS, _COLS), lambda i: (i, 0)),
        ],
        out_specs=[
            pl.BlockSpec((1, 1), lambda i: (0, 0)),
            pl.BlockSpec((1, 1), lambda i: (0, 0)),
        ],
        out_shape=[
            jax.ShapeDtypeStruct((1, 1), jnp.float32),
            jax.ShapeDtypeStruct((1, 1), jnp.float32),
        ],
        compiler_params=pltpu.CompilerParams(
            dimension_semantics=("arbitrary",),
        ),
    )(input, data_mask)
    return s[0, 0] / c[0, 0]
